# baseline (speedup 1.0000x reference)
import functools

import jax
import jax.numpy as jnp
from jax import lax
from jax.experimental import pallas as pl
from jax.experimental.pallas import tpu as pltpu
from jax.experimental.pallas import tpu_sc as plsc

N = 10000
E = 320000
D = 128
NC = 2
NS = 16
EW = E // (NC * NS)
CK = 80
NCH = EW // CK
NPA = 10112
RT = NPA // NS
NPD = 10240
RTD = NPD // NS
SLOTS = 3
NGRP = NCH // SLOTS

_mesh = plsc.VectorSubcoreMesh(
    core_axis_name="c", subcore_axis_name="s", num_cores=NC, num_subcores=NS
)


@functools.partial(
    pl.kernel,
    out_type=(jax.ShapeDtypeStruct((NPD,), jnp.float32),
              jax.ShapeDtypeStruct((NPD,), jnp.float32)),
    mesh=_mesh,
    scratch_types=[
        pltpu.VMEM((NCH, CK), jnp.int32),
        pltpu.VMEM((CK,), jnp.float32),
        pltpu.VMEM_SHARED((NPD,), jnp.float32),
        pltpu.SemaphoreType.DMA,
    ],
)
def _sc_degree(dst2_hbm, ones_hbm, zeros_hbm, deg0_hbm, deg1_hbm,
               dstv, ones_v, deg_sh, dsem):
    c = lax.axis_index("c")
    s = lax.axis_index("s")
    w = c * NS + s
    pltpu.sync_copy(zeros_hbm.at[pl.ds(s * RTD, RTD)], deg_sh.at[pl.ds(s * RTD, RTD)])
    pltpu.sync_copy(dst2_hbm.at[w], dstv)
    pltpu.sync_copy(ones_hbm, ones_v)
    plsc.subcore_barrier()

    def fire(j, carry):
        pltpu.async_copy(ones_v, deg_sh.at[dstv.at[j]], dsem, add=True)
        return carry

    lax.fori_loop(0, NCH, fire, 0)

    def drain(j, carry):
        pltpu.make_async_copy(ones_v, deg_sh.at[dstv.at[j]], dsem).wait()
        return carry

    lax.fori_loop(0, NCH, drain, 0)
    plsc.subcore_barrier()

    @pl.when(c == 0)
    def _():
        pltpu.sync_copy(deg_sh.at[pl.ds(s * RTD, RTD)], deg0_hbm.at[pl.ds(s * RTD, RTD)])

    @pl.when(c == 1)
    def _():
        pltpu.sync_copy(deg_sh.at[pl.ds(s * RTD, RTD)], deg1_hbm.at[pl.ds(s * RTD, RTD)])


@functools.partial(
    pl.kernel,
    out_type=jax.ShapeDtypeStruct((NC, NPA, D), jnp.float32),
    mesh=_mesh,
    scratch_types=[
        pltpu.VMEM((NCH, CK), jnp.int32),
        pltpu.VMEM((CK,), jnp.int32),
        pltpu.VMEM((CK,), jnp.int32),
        pltpu.VMEM((CK,), jnp.int32),
        pltpu.VMEM((CK, D), jnp.float32),
        pltpu.VMEM((CK, D), jnp.float32),
        pltpu.VMEM((CK, D), jnp.float32),
        pltpu.VMEM_SHARED((NPA, D), jnp.float32),
        pltpu.SemaphoreType.DMA,
        pltpu.SemaphoreType.DMA,
        pltpu.SemaphoreType.DMA,
        pltpu.SemaphoreType.DMA,
        pltpu.SemaphoreType.DMA,
        pltpu.SemaphoreType.DMA,
    ],
)
def _sc_agg(src2_hbm, dst_hbm, y_hbm, zeros_hbm, agg_hbm,
            srcv, di0, di1, di2, b0, b1, b2, agg_sh,
            g0, g1, g2, s0, s1, s2):
    c = lax.axis_index("c")
    s = lax.axis_index("s")
    w = c * NS + s
    base = w * EW
    didx = (di0, di1, di2)
    bufs = (b0, b1, b2)
    gsem = (g0, g1, g2)
    ssem = (s0, s1, s2)
    pltpu.sync_copy(zeros_hbm, agg_sh.at[pl.ds(s * RT, RT)])
    pltpu.sync_copy(src2_hbm.at[w], srcv)
    plsc.subcore_barrier()

    def fire(j, k):
        pltpu.async_copy(dst_hbm.at[pl.ds(base + j * CK, CK)], didx[k], gsem[k])
        pltpu.async_copy(y_hbm.at[srcv.at[j]], bufs[k], gsem[k])

    def wait_gather(j, k):
        pltpu.make_async_copy(
            dst_hbm.at[pl.ds(base + j * CK, CK)], didx[k], gsem[k]).wait()
        pltpu.make_async_copy(y_hbm.at[srcv.at[j]], bufs[k], gsem[k]).wait()

    for k in range(SLOTS):
        fire(k, k)

    def body(t, carry):
        j0 = t * SLOTS
        handles = []
        for k in range(SLOTS):
            wait_gather(j0 + k, k)
            handles.append(pltpu.async_copy(
                bufs[k], agg_sh.at[didx[k]], ssem[k], add=True))
        for k, h in enumerate(handles):
            h.wait()

            @pl.when(t < NGRP - 1)
            def _():
                fire(j0 + SLOTS + k, k)

        return carry

    lax.fori_loop(0, NGRP, body, 0)
    for i, j in enumerate(range(SLOTS * NGRP, NCH)):
        fire(j, i)
        wait_gather(j, i)
        pltpu.async_copy(bufs[i], agg_sh.at[didx[i]], ssem[i], add=True).wait()
    plsc.subcore_barrier()
    pltpu.sync_copy(agg_sh.at[pl.ds(s * RT, RT)], agg_hbm.at[c, pl.ds(s * RT, RT)])


_BR = 2000


def _tc_y_body(x_ref, d0_ref, d1_ref, y_ref):
    dis = lax.rsqrt(d0_ref[...] + d1_ref[...] + 1.0)
    y_ref[...] = x_ref[...] * dis


def _tc_y(x, d0, d1):
    return pl.pallas_call(
        _tc_y_body,
        grid=(N // _BR,),
        in_specs=[
            pl.BlockSpec((_BR, D), lambda i: (i, 0)),
            pl.BlockSpec((_BR, 1), lambda i: (i, 0)),
            pl.BlockSpec((_BR, 1), lambda i: (i, 0)),
        ],
        out_specs=pl.BlockSpec((_BR, D), lambda i: (i, 0)),
        out_shape=jax.ShapeDtypeStruct((N, D), jnp.float32),
    )(x, d0, d1)


def _tc_out_body(agg_ref, x_ref, d0_ref, d1_ref, wt_ref, b_ref, o_ref):
    dis = lax.rsqrt(d0_ref[...] + d1_ref[...] + 1.0)
    t = (agg_ref[0] + agg_ref[1]) * dis + x_ref[...] * (dis * dis)
    o_ref[...] = (
        jnp.dot(t, wt_ref[...], preferred_element_type=jnp.float32) + b_ref[...]
    )


def _tc_out(agg2, x, d0, d1, wt, b2):
    return pl.pallas_call(
        _tc_out_body,
        grid=(N // _BR,),
        in_specs=[
            pl.BlockSpec((NC, _BR, D), lambda i: (0, i, 0)),
            pl.BlockSpec((_BR, D), lambda i: (i, 0)),
            pl.BlockSpec((_BR, 1), lambda i: (i, 0)),
            pl.BlockSpec((_BR, 1), lambda i: (i, 0)),
            pl.BlockSpec((D, D), lambda i: (0, 0)),
            pl.BlockSpec((1, D), lambda i: (0, 0)),
        ],
        out_specs=pl.BlockSpec((_BR, D), lambda i: (i, 0)),
        out_shape=jax.ShapeDtypeStruct((N, D), jnp.float32),
    )(agg2, x, d0, d1, wt, b2)


def kernel(x, edge_index, W, b):
    src = edge_index[0].astype(jnp.int32)
    dst = edge_index[1].astype(jnp.int32)
    src2 = src.reshape(NC * NS, NCH, CK)
    dst2 = dst.reshape(NC * NS, NCH, CK)
    ones_c = jnp.ones((CK,), jnp.float32)
    zeros_d = jnp.zeros((NPD,), jnp.float32)
    zeros_a = jnp.zeros((RT, D), jnp.float32)

    deg0, deg1 = _sc_degree(dst2, ones_c, zeros_d)
    d0 = deg0[:N].reshape(N, 1)
    d1 = deg1[:N].reshape(N, 1)
    y = _tc_y(x, d0, d1)
    agg2 = _sc_agg(src2, dst, y, zeros_a)
    return _tc_out(agg2, x, d0, d1, W.T, b.reshape(1, D))


# speedup vs baseline: 1.0332x; 1.0332x over previous
import functools

import jax
import jax.numpy as jnp
from jax import lax
from jax.experimental import pallas as pl
from jax.experimental.pallas import tpu as pltpu
from jax.experimental.pallas import tpu_sc as plsc

N = 10000
E = 320000
D = 128
NC = 2
NS = 16
EW = E // (NC * NS)
CK = 80
NCH = EW // CK
NPA = 10112
RT = NPA // NS
NPD = 10240
RTD = NPD // NS
SLOTS = 3
NGRP = NCH // SLOTS

_mesh = plsc.VectorSubcoreMesh(
    core_axis_name="c", subcore_axis_name="s", num_cores=NC, num_subcores=NS
)


@functools.partial(
    pl.kernel,
    out_type=(jax.ShapeDtypeStruct((NPD,), jnp.float32),
              jax.ShapeDtypeStruct((NPD,), jnp.float32)),
    mesh=_mesh,
    scratch_types=[
        pltpu.VMEM((NCH, CK), jnp.int32),
        pltpu.VMEM((CK,), jnp.float32),
        pltpu.VMEM_SHARED((NPD,), jnp.float32),
        pltpu.SemaphoreType.DMA,
    ],
)
def _sc_degree(edge4_hbm, ones_hbm, zeros_hbm, deg0_hbm, deg1_hbm,
               dstv, ones_v, deg_sh, dsem):
    c = lax.axis_index("c")
    s = lax.axis_index("s")
    w = c * NS + s
    pltpu.sync_copy(zeros_hbm.at[pl.ds(s * RTD, RTD)], deg_sh.at[pl.ds(s * RTD, RTD)])
    pltpu.sync_copy(edge4_hbm.at[1, w], dstv)
    pltpu.sync_copy(ones_hbm, ones_v)
    plsc.subcore_barrier()

    def fire(j, carry):
        pltpu.async_copy(ones_v, deg_sh.at[dstv.at[j]], dsem, add=True)
        return carry

    lax.fori_loop(0, NCH, fire, 0)

    def drain(j, carry):
        pltpu.make_async_copy(ones_v, deg_sh.at[dstv.at[j]], dsem).wait()
        return carry

    lax.fori_loop(0, NCH, drain, 0)
    plsc.subcore_barrier()

    @pl.when(c == 0)
    def _():
        pltpu.sync_copy(deg_sh.at[pl.ds(s * RTD, RTD)], deg0_hbm.at[pl.ds(s * RTD, RTD)])

    @pl.when(c == 1)
    def _():
        pltpu.sync_copy(deg_sh.at[pl.ds(s * RTD, RTD)], deg1_hbm.at[pl.ds(s * RTD, RTD)])


@functools.partial(
    pl.kernel,
    out_type=jax.ShapeDtypeStruct((NC, NPA, D), jnp.float32),
    mesh=_mesh,
    scratch_types=[
        pltpu.VMEM((NCH, CK), jnp.int32),
        pltpu.VMEM((CK,), jnp.int32),
        pltpu.VMEM((CK,), jnp.int32),
        pltpu.VMEM((CK,), jnp.int32),
        pltpu.VMEM((CK, D), jnp.float32),
        pltpu.VMEM((CK, D), jnp.float32),
        pltpu.VMEM((CK, D), jnp.float32),
        pltpu.VMEM_SHARED((NPA, D), jnp.float32),
        pltpu.SemaphoreType.DMA,
        pltpu.SemaphoreType.DMA,
        pltpu.SemaphoreType.DMA,
        pltpu.SemaphoreType.DMA,
        pltpu.SemaphoreType.DMA,
        pltpu.SemaphoreType.DMA,
    ],
)
def _sc_agg(edge4_hbm, dst_hbm, y_hbm, zeros_hbm, agg_hbm,
            srcv, di0, di1, di2, b0, b1, b2, agg_sh,
            g0, g1, g2, s0, s1, s2):
    c = lax.axis_index("c")
    s = lax.axis_index("s")
    w = c * NS + s
    base = w * EW
    didx = (di0, di1, di2)
    bufs = (b0, b1, b2)
    gsem = (g0, g1, g2)
    ssem = (s0, s1, s2)
    pltpu.sync_copy(zeros_hbm, agg_sh.at[pl.ds(s * RT, RT)])
    pltpu.sync_copy(edge4_hbm.at[0, w], srcv)
    plsc.subcore_barrier()

    def fire(j, k):
        pltpu.async_copy(dst_hbm.at[pl.ds(base + j * CK, CK)], didx[k], gsem[k])
        pltpu.async_copy(y_hbm.at[srcv.at[j]], bufs[k], gsem[k])

    def wait_gather(j, k):
        pltpu.make_async_copy(
            dst_hbm.at[pl.ds(base + j * CK, CK)], didx[k], gsem[k]).wait()
        pltpu.make_async_copy(y_hbm.at[srcv.at[j]], bufs[k], gsem[k]).wait()

    for k in range(SLOTS):
        fire(k, k)

    def body(t, carry):
        j0 = t * SLOTS
        handles = []
        for k in range(SLOTS):
            wait_gather(j0 + k, k)
            handles.append(pltpu.async_copy(
                bufs[k], agg_sh.at[didx[k]], ssem[k], add=True))
        for k, h in enumerate(handles):
            h.wait()

            @pl.when(t < NGRP - 1)
            def _():
                fire(j0 + SLOTS + k, k)

        return carry

    lax.fori_loop(0, NGRP, body, 0)
    for i, j in enumerate(range(SLOTS * NGRP, NCH)):
        fire(j, i)
        wait_gather(j, i)
        pltpu.async_copy(bufs[i], agg_sh.at[didx[i]], ssem[i], add=True).wait()
    plsc.subcore_barrier()
    pltpu.sync_copy(agg_sh.at[pl.ds(s * RT, RT)], agg_hbm.at[c, pl.ds(s * RT, RT)])


_BR = 2000


def _tc_y_body(x_ref, d0_ref, d1_ref, y_ref):
    dis = lax.rsqrt(d0_ref[...] + d1_ref[...] + 1.0)
    y_ref[...] = x_ref[...] * dis


def _tc_y(x, d0, d1):
    return pl.pallas_call(
        _tc_y_body,
        grid=(N // _BR,),
        in_specs=[
            pl.BlockSpec((_BR, D), lambda i: (i, 0)),
            pl.BlockSpec((_BR, 1), lambda i: (i, 0)),
            pl.BlockSpec((_BR, 1), lambda i: (i, 0)),
        ],
        out_specs=pl.BlockSpec((_BR, D), lambda i: (i, 0)),
        out_shape=jax.ShapeDtypeStruct((N, D), jnp.float32),
    )(x, d0, d1)


def _tc_out_body(agg_ref, x_ref, d0_ref, d1_ref, wt_ref, b_ref, o_ref):
    dis = lax.rsqrt(d0_ref[...] + d1_ref[...] + 1.0)
    t = (agg_ref[0] + agg_ref[1]) * dis + x_ref[...] * (dis * dis)
    o_ref[...] = (
        jnp.dot(t, wt_ref[...], preferred_element_type=jnp.float32) + b_ref[...]
    )


def _tc_out(agg2, x, d0, d1, wt, b2):
    return pl.pallas_call(
        _tc_out_body,
        grid=(N // _BR,),
        in_specs=[
            pl.BlockSpec((NC, _BR, D), lambda i: (0, i, 0)),
            pl.BlockSpec((_BR, D), lambda i: (i, 0)),
            pl.BlockSpec((_BR, 1), lambda i: (i, 0)),
            pl.BlockSpec((_BR, 1), lambda i: (i, 0)),
            pl.BlockSpec((D, D), lambda i: (0, 0)),
            pl.BlockSpec((1, D), lambda i: (0, 0)),
        ],
        out_specs=pl.BlockSpec((_BR, D), lambda i: (i, 0)),
        out_shape=jax.ShapeDtypeStruct((N, D), jnp.float32),
    )(agg2, x, d0, d1, wt, b2)


def kernel(x, edge_index, W, b):
    edge4 = edge_index.astype(jnp.int32).reshape(2, NC * NS, NCH, CK)
    dst = edge_index[1].astype(jnp.int32)
    ones_c = jnp.ones((CK,), jnp.float32)
    zeros_d = jnp.zeros((NPD,), jnp.float32)
    zeros_a = jnp.zeros((RT, D), jnp.float32)

    deg0, deg1 = _sc_degree(edge4, ones_c, zeros_d)
    d0 = deg0.reshape(NPD, 1)
    d1 = deg1.reshape(NPD, 1)
    y = _tc_y(x, d0, d1)
    agg2 = _sc_agg(edge4, dst, y, zeros_a)
    return _tc_out(agg2, x, d0, d1, W.T, b.reshape(1, D))
